# Initial kernel scaffold; baseline (speedup 1.0000x reference)
import jax
import jax.numpy as jnp
from jax.experimental import pallas as pl


def kernel(X, Z):
    raise NotImplementedError("write your pallas kernel here")


# speedup vs baseline: 13.1369x; 13.1369x over previous
import jax
import jax.numpy as jnp
from jax.experimental import pallas as pl
from jax.experimental.pallas import tpu as pltpu

_N = 4096
_R = 256
_KNN = 8
_INF = float("inf")


def _dist_block(rows, alln, diag, d_ref):
    g = jax.lax.dot_general(rows, alln, (((1,), (1,)), ((), ())),
                            preferred_element_type=jnp.float32)
    sq_r = jnp.sum(rows * rows, axis=1, keepdims=True)
    ones = jnp.ones((1, rows.shape[1]), jnp.float32)
    sq_a = jax.lax.dot_general(ones, alln * alln, (((1,), (1,)), ((), ())),
                               preferred_element_type=jnp.float32)
    d_ref[...] = jnp.where(diag, _INF, (sq_r + sq_a) - 2.0 * g)


def _mask_topk(d_ref, cols):
    for _ in range(_KNN):
        d = d_ref[...]
        m = jnp.min(d, axis=1, keepdims=True)
        idx = jnp.min(jnp.where(d == m, cols, _N), axis=1, keepdims=True)
        d_ref[...] = jnp.where(cols == idx, _INF, d)


def _body(xr, xa, zr, za, out_ref, dx_ref, dz_ref):
    i = pl.program_id(0)
    cols = jax.lax.broadcasted_iota(jnp.int32, (_R, _N), 1)
    row_g = i * _R + jax.lax.broadcasted_iota(jnp.int32, (_R, _N), 0)
    diag = cols == row_g

    _dist_block(xr[...], xa[...], diag, dx_ref)
    _mask_topk(dx_ref, cols)

    _dist_block(zr[...], za[...], diag, dz_ref)
    _mask_topk(dz_ref, cols)

    both = (dx_ref[...] == _INF) & (dz_ref[...] == _INF) & jnp.logical_not(diag)
    c = jnp.sum(both.astype(jnp.float32))
    partial = (16.0 * _R - 2.0 * c) * (100.0 / (_N * _N))

    @pl.when(i == 0)
    def _init():
        out_ref[...] = jnp.zeros((1, 1), jnp.float32)

    out_ref[...] = out_ref[...] + partial


def kernel(X, Z):
    n, dx = X.shape
    _, dz = Z.shape
    out = pl.pallas_call(
        _body,
        grid=(n // _R,),
        in_specs=[
            pl.BlockSpec((_R, dx), lambda i: (i, 0)),
            pl.BlockSpec((n, dx), lambda i: (0, 0)),
            pl.BlockSpec((_R, dz), lambda i: (i, 0)),
            pl.BlockSpec((n, dz), lambda i: (0, 0)),
        ],
        out_specs=pl.BlockSpec((1, 1), lambda i: (0, 0)),
        out_shape=jax.ShapeDtypeStruct((1, 1), jnp.float32),
        scratch_shapes=[
            pltpu.VMEM((_R, _N), jnp.float32),
            pltpu.VMEM((_R, _N), jnp.float32),
        ],
    )(X, X, Z, Z)
    return out[0, 0]


# speedup vs baseline: 13.2001x; 1.0048x over previous
import jax
import jax.numpy as jnp
from jax.experimental import pallas as pl
from jax.experimental.pallas import tpu as pltpu

_N = 4096
_R = 256
_KNN = 8
_INF = float("inf")


def _dist_block(rows, alln, diag, d_ref):
    g = jax.lax.dot_general(rows, alln, (((1,), (1,)), ((), ())),
                            preferred_element_type=jnp.float32)
    sq_r = jnp.sum(rows * rows, axis=1, keepdims=True)
    ones = jnp.ones((1, rows.shape[1]), jnp.float32)
    sq_a = jax.lax.dot_general(ones, alln * alln, (((1,), (1,)), ((), ())),
                               preferred_element_type=jnp.float32)
    d_ref[...] = jnp.where(diag, _INF, (sq_r + sq_a) - 2.0 * g)


def _mask_topk(d_ref, cols):
    idx = jnp.argmin(d_ref[...], axis=1, keepdims=True)
    for _ in range(_KNN - 1):
        d = jnp.where(cols == idx, _INF, d_ref[...])
        d_ref[...] = d
        idx = jnp.argmin(d, axis=1, keepdims=True)
    return idx


def _body(xr, xa, zr, za, out_ref, dx_ref, dz_ref):
    i = pl.program_id(0)
    cols = jax.lax.broadcasted_iota(jnp.int32, (_R, _N), 1)
    row_g = i * _R + jax.lax.broadcasted_iota(jnp.int32, (_R, _N), 0)
    diag = cols == row_g

    _dist_block(xr[...], xa[...], diag, dx_ref)
    ix_last = _mask_topk(dx_ref, cols)

    _dist_block(zr[...], za[...], diag, dz_ref)
    iz_last = _mask_topk(dz_ref, cols)

    sel_x = (dx_ref[...] == _INF) | (cols == ix_last)
    sel_z = (dz_ref[...] == _INF) | (cols == iz_last)
    both = sel_x & sel_z & jnp.logical_not(diag)
    c = jnp.sum(both.astype(jnp.float32))
    partial = (16.0 * _R - 2.0 * c) * (100.0 / (_N * _N))

    @pl.when(i == 0)
    def _init():
        out_ref[...] = jnp.zeros((1, 1), jnp.float32)

    out_ref[...] = out_ref[...] + partial


def kernel(X, Z):
    n, dx = X.shape
    _, dz = Z.shape
    out = pl.pallas_call(
        _body,
        grid=(n // _R,),
        in_specs=[
            pl.BlockSpec((_R, dx), lambda i: (i, 0)),
            pl.BlockSpec((n, dx), lambda i: (0, 0)),
            pl.BlockSpec((_R, dz), lambda i: (i, 0)),
            pl.BlockSpec((n, dz), lambda i: (0, 0)),
        ],
        out_specs=pl.BlockSpec((1, 1), lambda i: (0, 0)),
        out_shape=jax.ShapeDtypeStruct((1, 1), jnp.float32),
        scratch_shapes=[
            pltpu.VMEM((_R, _N), jnp.float32),
            pltpu.VMEM((_R, _N), jnp.float32),
        ],
    )(X, X, Z, Z)
    return out[0, 0]
